# baseline (speedup 1.0000x reference)
import functools

import jax
import jax.numpy as jnp
from jax import lax
from jax.experimental import pallas as pl
from jax.experimental.pallas import tpu as pltpu
from jax.experimental.pallas import tpu_sc as plsc

N = 50000
E = 800000
NR = 3
NG = 128
HID = 64
NHALF = N // 2
CNT_ROWS = NR * NHALF
CNT_PAD = 75008
CNT_SL = CNT_PAD // 16
ACC_PAD = 50048
ACC_SL = ACC_PAD // 16
ACC_ST = 184
TE = 51200
EP = 16 * TE
SUB = 16
CHUNK = SUB * 128
NCH = TE // CHUNK
SUBE = 4
CHUNKE = SUBE * 128
NCHE = TE // CHUNKE
NB = 2000
NBLK = N // NB

_mesh = plsc.VectorSubcoreMesh(core_axis_name="c", subcore_axis_name="s")


@functools.partial(
    pl.kernel,
    out_type=[
        jax.ShapeDtypeStruct((EP + 128,), jnp.float32),
        jax.ShapeDtypeStruct((EP // 128, 128), jnp.int32),
    ],
    mesh=_mesh,
    scratch_types=[
        pltpu.VMEM_SHARED((CNT_PAD,), jnp.float32),
        pltpu.VMEM((CHUNK,), jnp.int32),
        pltpu.VMEM((CHUNK,), jnp.int32),
        pltpu.VMEM((CHUNK,), jnp.int32),
        pltpu.VMEM((SUB, 128), jnp.int32),
        pltpu.VMEM((SUB, 128), jnp.int32),
        pltpu.VMEM((SUB, 128), jnp.int32),
        pltpu.VMEM((SUB, 128), jnp.float32),
        pltpu.VMEM((128,), jnp.float32),
        pltpu.VMEM((CNT_SL,), jnp.float32),
        pltpu.SemaphoreType.DMA,
    ],
    compiler_params=pltpu.CompilerParams(use_tc_tiling_on_sc=False),
)
def _sc_count(dst_h, et_h, src_h, w_h, g_h,
              cnt_sh, dbuf, ebuf, sbuf, ibuf, pbuf, gbuf, wbuf, obuf, cbuf,
              sem):
    cid = lax.axis_index("c")
    sid = lax.axis_index("s")
    lo = cid * NHALF

    def zfill(i, _):
        cbuf[pl.ds(i * 16, 16)] = jnp.zeros((16,), jnp.float32)
        return 0

    lax.fori_loop(0, CNT_SL // 16, zfill, 0)
    pltpu.sync_copy(cbuf, cnt_sh.at[pl.ds(sid * CNT_SL, CNT_SL)])
    for i in range(8):
        obuf[pl.ds(i * 16, 16)] = jnp.ones((16,), jnp.float32)
    plsc.subcore_barrier()

    def count_chunk(ch, _):
        base = sid * TE + ch * CHUNK
        pltpu.sync_copy(dst_h.at[pl.ds(base, CHUNK)], dbuf)
        pltpu.sync_copy(et_h.at[pl.ds(base, CHUNK)], ebuf)
        for i in range(CHUNK // 16):
            d = dbuf[pl.ds(i * 16, 16)]
            e = ebuf[pl.ds(i * 16, 16)]
            owned = (d >= lo) & (d < lo + NHALF)
            loc = e * NHALF + (d - lo)
            idx = jnp.where(owned, loc, CNT_ROWS)
            ibuf[i // 8, pl.ds((i % 8) * 16, 16)] = idx
        for s in range(SUB):
            pltpu.sync_copy(obuf, cnt_sh.at[ibuf.at[s]], add=True)
        return 0

    lax.fori_loop(0, NCH, count_chunk, 0)
    plsc.subcore_barrier()

    pltpu.sync_copy(cnt_sh.at[pl.ds(sid * CNT_SL, CNT_SL)], cbuf)

    def inv_step(i, _):
        v = cbuf[pl.ds(i * 16, 16)]
        cbuf[pl.ds(i * 16, 16)] = 1.0 / jnp.maximum(v, 1.0)
        return 0

    lax.fori_loop(0, CNT_SL // 16, inv_step, 0)
    pltpu.sync_copy(cbuf, cnt_sh.at[pl.ds(sid * CNT_SL, CNT_SL)])
    plsc.subcore_barrier()

    def w_chunk(ch, _):
        base = sid * TE + ch * CHUNK
        pltpu.sync_copy(dst_h.at[pl.ds(base, CHUNK)], dbuf)
        pltpu.sync_copy(et_h.at[pl.ds(base, CHUNK)], ebuf)
        pltpu.sync_copy(src_h.at[pl.ds(base, CHUNK)], sbuf)
        for i in range(CHUNK // 16):
            d = dbuf[pl.ds(i * 16, 16)]
            e = ebuf[pl.ds(i * 16, 16)]
            sv = sbuf[pl.ds(i * 16, 16)]
            owned = (d >= lo) & (d < lo + NHALF)
            loc = e * NHALF + (d - lo)
            s, j = i // 8, (i % 8) * 16
            ibuf[s, pl.ds(j, 16)] = jnp.where(owned, loc, CNT_ROWS)
            pos = base + i * 16 + lax.broadcasted_iota(jnp.int32, (16,), 0)
            pbuf[s, pl.ds(j, 16)] = jnp.where(owned, pos, EP)
            gbuf[s, pl.ds(j, 16)] = sv * NR + e
        descs = [pltpu.async_copy(cnt_sh.at[ibuf.at[s]], wbuf.at[s], sem)
                 for s in range(SUB)]
        for dd in descs:
            dd.wait()
        for s in range(SUB):
            pltpu.sync_copy(wbuf.at[s], w_h.at[pbuf.at[s]])

        @pl.when(cid == 0)
        def _():
            pltpu.sync_copy(gbuf, g_h.at[pl.ds(sid * (TE // 128) + ch * SUB,
                                               SUB)])

        return 0

    lax.fori_loop(0, NCH, w_chunk, 0)


@functools.partial(
    pl.kernel,
    out_type=jax.ShapeDtypeStruct((2 * ACC_PAD, 32), jnp.float32),
    mesh=_mesh,
    scratch_types=[
        pltpu.VMEM_SHARED((ACC_PAD, 32), jnp.float32),
        pltpu.VMEM((SUBE, 128), jnp.int32),
        pltpu.VMEM((SUBE, 128), jnp.int32),
        pltpu.VMEM((CHUNKE,), jnp.float32),
        pltpu.VMEM((SUBE, 128, 32), jnp.float32),
        pltpu.VMEM((ACC_ST, 32), jnp.float32),
        pltpu.SemaphoreType.DMA,
    ],
    compiler_params=pltpu.CompilerParams(use_tc_tiling_on_sc=False),
)
def _sc_edge(tlo_h, thi_h, g_h, dst_h, w_h, acc_h,
             acc_sh, gb, db, wb, rows, stg, sem):
    cid = lax.axis_index("c")
    sid = lax.axis_index("s")

    def zfill(i, _):
        stg[i, pl.ds(0, 16)] = jnp.zeros((16,), jnp.float32)
        stg[i, pl.ds(16, 16)] = jnp.zeros((16,), jnp.float32)
        return 0

    lax.fori_loop(0, ACC_ST, zfill, 0)
    for t in range(ACC_SL // ACC_ST):
        pltpu.sync_copy(stg, acc_sh.at[pl.ds(sid * ACC_SL + t * ACC_ST,
                                             ACC_ST)])
    plsc.subcore_barrier()

    def edge_chunk(ch, _):
        rowbase = sid * (TE // 128) + ch * SUBE
        base = sid * TE + ch * CHUNKE
        pltpu.sync_copy(g_h.at[pl.ds(rowbase, SUBE)], gb)
        pltpu.sync_copy(dst_h.at[pl.ds(rowbase, SUBE)], db)
        pltpu.sync_copy(w_h.at[pl.ds(base, CHUNKE)], wb)

        @pl.when(cid == 0)
        def _():
            descs = [pltpu.async_copy(tlo_h.at[gb.at[s]], rows.at[s], sem)
                     for s in range(SUBE)]
            for dd in descs:
                dd.wait()

        @pl.when(cid == 1)
        def _():
            descs = [pltpu.async_copy(thi_h.at[gb.at[s]], rows.at[s], sem)
                     for s in range(SUBE)]
            for dd in descs:
                dd.wait()

        def scale(i, _):
            wv = wb[pl.ds(i * 16, 16)]
            for k in range(16):
                e = i * 16 + k
                s = e // 128
                j = e - s * 128
                w = wv[k]
                rows[s, j, pl.ds(0, 16)] = rows[s, j, pl.ds(0, 16)] * w
                rows[s, j, pl.ds(16, 16)] = rows[s, j, pl.ds(16, 16)] * w
            return 0

        lax.fori_loop(0, CHUNKE // 16, scale, 0)
        for s in range(SUBE):
            pltpu.sync_copy(rows.at[s], acc_sh.at[db.at[s]], add=True)
        return 0

    lax.fori_loop(0, NCHE, edge_chunk, 0)
    plsc.subcore_barrier()

    for t in range(ACC_SL // ACC_ST):
        pltpu.sync_copy(acc_sh.at[pl.ds(sid * ACC_SL + t * ACC_ST, ACC_ST)],
                        stg)
        pltpu.sync_copy(stg, acc_h.at[pl.ds(cid * ACC_PAD + sid * ACC_SL
                                            + t * ACC_ST, ACC_ST)])


def _tc_tables(ecat, b_lo, b_hi, b_u):
    def body(e_ref, lo_ref, hi_ref, u_ref, tlo_ref, thi_ref, tu_ref):
        e = e_ref[...]
        tlo_ref[...] = jnp.dot(e, lo_ref[...], preferred_element_type=jnp.float32)
        thi_ref[...] = jnp.dot(e, hi_ref[...], preferred_element_type=jnp.float32)
        tu_ref[...] = jnp.dot(e, u_ref[...], preferred_element_type=jnp.float32)

    return pl.pallas_call(
        body,
        out_shape=[
            jax.ShapeDtypeStruct((160, 96), jnp.float32),
            jax.ShapeDtypeStruct((160, 96), jnp.float32),
            jax.ShapeDtypeStruct((160, 64), jnp.float32),
        ],
    )(ecat, b_lo, b_hi, b_u)


def _tc_layer1(x, t_lo, t_hi, t_u, b1):
    def body(x_ref, tlo_ref, thi_ref, tu_ref, b_ref, olo_ref, ohi_ref, ou_ref):
        xb = x_ref[...]
        cols = lax.broadcasted_iota(jnp.int32, (NB, 160), 1)
        oh = ((cols == xb[:, 0:1]) | (cols == xb[:, 1:2] + 16)
              | (cols == xb[:, 2:3] + 32)).astype(jnp.float32)
        olo_ref[...] = jnp.dot(oh, tlo_ref[...], preferred_element_type=jnp.float32)
        ohi_ref[...] = jnp.dot(oh, thi_ref[...], preferred_element_type=jnp.float32)
        ou_ref[...] = jnp.dot(oh, tu_ref[...],
                              preferred_element_type=jnp.float32) + b_ref[...]

    return pl.pallas_call(
        body,
        grid=(NBLK,),
        in_specs=[
            pl.BlockSpec((NB, 3), lambda i: (i, 0)),
            pl.BlockSpec((160, 96), lambda i: (0, 0)),
            pl.BlockSpec((160, 96), lambda i: (0, 0)),
            pl.BlockSpec((160, 64), lambda i: (0, 0)),
            pl.BlockSpec((1, 64), lambda i: (0, 0)),
        ],
        out_specs=[
            pl.BlockSpec((NB, 96), lambda i: (i, 0)),
            pl.BlockSpec((NB, 96), lambda i: (i, 0)),
            pl.BlockSpec((NB, 64), lambda i: (i, 0)),
        ],
        out_shape=[
            jax.ShapeDtypeStruct((N, 96), jnp.float32),
            jax.ShapeDtypeStruct((N, 96), jnp.float32),
            jax.ShapeDtypeStruct((N, 64), jnp.float32),
        ],
    )(x, t_lo, t_hi, t_u, b1)


def _tc_layer2(u1, acc1, b_lo, b_hi, b_u, b2):
    def body(u_ref, a_ref, lo_ref, hi_ref, bu_ref, b2_ref,
             olo_ref, ohi_ref, ou_ref):
        h = jnp.maximum(u_ref[...] + a_ref[...], 0.0)
        olo_ref[...] = jnp.dot(h, lo_ref[...], preferred_element_type=jnp.float32)
        ohi_ref[...] = jnp.dot(h, hi_ref[...], preferred_element_type=jnp.float32)
        ou_ref[...] = jnp.dot(h, bu_ref[...],
                              preferred_element_type=jnp.float32) + b2_ref[...]

    return pl.pallas_call(
        body,
        grid=(NBLK,),
        in_specs=[
            pl.BlockSpec((NB, 64), lambda i: (i, 0)),
            pl.BlockSpec((NB, 64), lambda i: (i, 0)),
            pl.BlockSpec((64, 96), lambda i: (0, 0)),
            pl.BlockSpec((64, 96), lambda i: (0, 0)),
            pl.BlockSpec((64, 64), lambda i: (0, 0)),
            pl.BlockSpec((1, 64), lambda i: (0, 0)),
        ],
        out_specs=[
            pl.BlockSpec((NB, 96), lambda i: (i, 0)),
            pl.BlockSpec((NB, 96), lambda i: (i, 0)),
            pl.BlockSpec((NB, 64), lambda i: (i, 0)),
        ],
        out_shape=[
            jax.ShapeDtypeStruct((N, 96), jnp.float32),
            jax.ShapeDtypeStruct((N, 96), jnp.float32),
            jax.ShapeDtypeStruct((N, 64), jnp.float32),
        ],
    )(u1, acc1, b_lo, b_hi, b_u, b2)


def _tc_pool(u2, acc2, batch3):
    def body(u_ref, a_ref, b_ref, pool_ref, cnt_ref):
        @pl.when(pl.program_id(0) == 0)
        def _():
            pool_ref[...] = jnp.zeros_like(pool_ref)
            cnt_ref[...] = jnp.zeros_like(cnt_ref)

        h = jnp.maximum(u_ref[...] + a_ref[...], 0.0)
        b = b_ref[0]
        oht = (lax.broadcasted_iota(jnp.int32, (NG, NB), 0)
               == b).astype(jnp.float32)
        pool_ref[...] += jnp.dot(oht, h, preferred_element_type=jnp.float32)
        cnt_ref[...] += jnp.dot(oht, jnp.ones((NB, 8), jnp.float32),
                                preferred_element_type=jnp.float32)

    return pl.pallas_call(
        body,
        grid=(NBLK,),
        in_specs=[
            pl.BlockSpec((NB, 64), lambda i: (i, 0)),
            pl.BlockSpec((NB, 64), lambda i: (i, 0)),
            pl.BlockSpec((1, 1, NB), lambda i: (i, 0, 0)),
        ],
        out_specs=[
            pl.BlockSpec((NG, 64), lambda i: (0, 0)),
            pl.BlockSpec((NG, 8), lambda i: (0, 0)),
        ],
        out_shape=[
            jax.ShapeDtypeStruct((NG, 64), jnp.float32),
            jax.ShapeDtypeStruct((NG, 8), jnp.float32),
        ],
        compiler_params=pltpu.CompilerParams(
            dimension_semantics=("arbitrary",)),
    )(u2, acc2, batch3)


def _tc_head(pool, cnt, lin_W, lin_b):
    def body(p_ref, c_ref, w_ref, b_ref, o_ref):
        inv = 1.0 / jnp.maximum(c_ref[...][:, 0:1], 1.0)
        hg = p_ref[...] * inv
        o_ref[...] = jnp.dot(hg, w_ref[...],
                             preferred_element_type=jnp.float32) + b_ref[...]

    return pl.pallas_call(
        body,
        out_shape=jax.ShapeDtypeStruct((NG, 10), jnp.float32),
    )(pool, cnt, lin_W, lin_b)


def kernel(x, edge_index, edge_type, batch, shape_emb, color_emb, pos_emb,
           W1, root1, b1, W2, root2, b2, lin_W, lin_b):
    f32 = jnp.float32
    src = edge_index[0]
    dst = edge_index[1]

    pad = EP - E
    dst_p = jnp.concatenate([dst, jnp.full((pad,), N, jnp.int32)])
    et_p = jnp.concatenate([edge_type, jnp.zeros((pad,), jnp.int32)])
    src_p = jnp.concatenate([src, jnp.zeros((pad,), jnp.int32)])

    ecat = jnp.zeros((160, 80), f32)
    ecat = ecat.at[0:16, 0:32].set(shape_emb)
    ecat = ecat.at[16:32, 32:64].set(color_emb)
    ecat = ecat.at[32:160, 64:80].set(pos_emb)
    b1_lo = jnp.concatenate([W1[r][:, :32] for r in range(NR)], axis=1)
    b1_hi = jnp.concatenate([W1[r][:, 32:] for r in range(NR)], axis=1)
    t_lo, t_hi, t_u = _tc_tables(ecat, b1_lo, b1_hi, root1)

    w_p, g_p = _sc_count(dst_p, et_p, src_p)

    t1lo, t1hi, u1 = _tc_layer1(x, t_lo, t_hi, t_u, b1.reshape(1, HID))
    acc1 = _sc_edge(t1lo.reshape(NR * N, 32), t1hi.reshape(NR * N, 32),
                    g_p, dst_p.reshape(EP // 128, 128), w_p)
    acc1c = jnp.concatenate([acc1[:N], acc1[ACC_PAD:ACC_PAD + N]], axis=1)

    b2_lo = jnp.concatenate([W2[r][:, :32] for r in range(NR)], axis=1)
    b2_hi = jnp.concatenate([W2[r][:, 32:] for r in range(NR)], axis=1)
    t2lo, t2hi, u2 = _tc_layer2(u1, acc1c, b2_lo, b2_hi, root2,
                                b2.reshape(1, HID))
    acc2 = _sc_edge(t2lo.reshape(NR * N, 32), t2hi.reshape(NR * N, 32),
                    g_p, dst_p.reshape(EP // 128, 128), w_p)
    acc2c = jnp.concatenate([acc2[:N], acc2[ACC_PAD:ACC_PAD + N]], axis=1)

    pool, cnt = _tc_pool(u2, acc2c, batch.reshape(NBLK, 1, NB))
    return _tc_head(pool, cnt, lin_W, lin_b.reshape(1, 10))


# speedup vs baseline: 32.5222x; 32.5222x over previous
import functools

import jax
import jax.numpy as jnp
from jax import lax
from jax.experimental import pallas as pl
from jax.experimental.pallas import tpu as pltpu
from jax.experimental.pallas import tpu_sc as plsc

N = 50000
E = 800000
NR = 3
NG = 128
HID = 64
NHALF = N // 2
CNT_ROWS = NR * NHALF
CNT_PAD = 75008
CNT_SL = CNT_PAD // 16
ACC_PAD = 50048
ACC_SL = ACC_PAD // 16
ACC_ST = 184
TE = 51200
EP = 16 * TE
SUBC = 8
CHUNKC = SUBC * 128
NCHC = TE // CHUNKC
DUMMYC = 50040
CST = 136
SUBE = 4
CHUNKE = SUBE * 128
NCHE = TE // CHUNKE
NB = 2000
NBLK = N // NB

_mesh = plsc.VectorSubcoreMesh(core_axis_name="c", subcore_axis_name="s")


@functools.partial(
    pl.kernel,
    out_type=[
        jax.ShapeDtypeStruct((EP + 128,), jnp.float32),
        jax.ShapeDtypeStruct((EP // 128, 128), jnp.int32),
    ],
    mesh=_mesh,
    scratch_types=[
        pltpu.VMEM_SHARED((ACC_PAD, 32), jnp.float32),
        pltpu.VMEM((SUBC, 128), jnp.int32),
        pltpu.VMEM((SUBC, 128), jnp.int32),
        pltpu.VMEM((SUBC, 128), jnp.int32),
        pltpu.VMEM((SUBC, 128), jnp.int32),
        pltpu.VMEM((SUBC, 128), jnp.int32),
        pltpu.VMEM((128, 32), jnp.float32),
        pltpu.VMEM((128, 32), jnp.float32),
        pltpu.VMEM((128, 32), jnp.float32),
        pltpu.VMEM((128, 32), jnp.float32),
        pltpu.VMEM((CHUNKC,), jnp.float32),
        pltpu.VMEM((CST, 32), jnp.float32),
    ],
    compiler_params=pltpu.CompilerParams(use_tc_tiling_on_sc=False,
                                         needs_layout_passes=False),
)
def _sc_count(dst_h, et_h, src_h, w_h, g_h,
              cnt_sh, db, eb, sb, ib, gb, v0, v1, v2, crows, wlin, stg):
    cid = lax.axis_index("c")
    sid = lax.axis_index("s")

    @pl.when(cid == 0)
    def _():
        def zfill(i, _):
            z = jnp.zeros((16,), jnp.float32)
            stg[i, pl.ds(0, 16)] = z
            stg[i, pl.ds(16, 16)] = z
            return 0

        lax.fori_loop(0, CST, zfill, 0)
        for t in range(ACC_SL // CST):
            pltpu.sync_copy(stg, cnt_sh.at[pl.ds(sid * ACC_SL + t * CST,
                                                 CST)])
        ohz = jnp.zeros((16,), jnp.float32)
        iot = lax.broadcasted_iota(jnp.int32, (16,), 0)

        def vfill(i, _):
            for r, vr in ((0, v0), (1, v1), (2, v2)):
                vr[i, pl.ds(0, 16)] = jnp.where(iot == r, 1.0, 0.0)
                vr[i, pl.ds(16, 16)] = ohz
            return 0

        lax.fori_loop(0, 128, vfill, 0)
        plsc.subcore_barrier()

        def count_chunk(ch, _):
            rowbase = sid * (TE // 128) + ch * SUBC
            pltpu.sync_copy(dst_h.at[pl.ds(rowbase, SUBC)], db)
            pltpu.sync_copy(et_h.at[pl.ds(rowbase, SUBC)], eb)
            for r, vr in ((0, v0), (1, v1), (2, v2)):
                for i in range(CHUNKC // 16):
                    s, j = i // 8, (i % 8) * 16
                    d = db[s, pl.ds(j, 16)]
                    e = eb[s, pl.ds(j, 16)]
                    ib[s, pl.ds(j, 16)] = jnp.where(e == r, d, DUMMYC)
                for s in range(SUBC):
                    pltpu.sync_copy(vr, cnt_sh.at[ib.at[s]], add=True)
            return 0

        lax.fori_loop(0, NCHC, count_chunk, 0)
        plsc.subcore_barrier()

        def inv_blk(t, _):
            nb = sid * ACC_SL + t * CST
            pltpu.sync_copy(cnt_sh.at[pl.ds(nb, CST)], stg)

            def inv_row(u, _):
                v = stg[u, pl.ds(0, 16)]
                stg[u, pl.ds(0, 16)] = 1.0 / jnp.maximum(v, 1.0)
                return 0

            lax.fori_loop(0, CST, inv_row, 0)
            pltpu.sync_copy(stg, cnt_sh.at[pl.ds(nb, CST)])
            return 0

        lax.fori_loop(0, ACC_SL // CST, inv_blk, 0)
        plsc.subcore_barrier()

        def w_chunk(ch, _):
            rowbase = sid * (TE // 128) + ch * SUBC
            base = sid * TE + ch * CHUNKC
            pltpu.sync_copy(dst_h.at[pl.ds(rowbase, SUBC)], db)
            pltpu.sync_copy(et_h.at[pl.ds(rowbase, SUBC)], eb)
            pltpu.sync_copy(src_h.at[pl.ds(rowbase, SUBC)], sb)
            for s in range(SUBC):
                pltpu.sync_copy(cnt_sh.at[db.at[s]], crows)
                for i in range(8):
                    j = i * 16
                    e = eb[s, pl.ds(j, 16)]
                    sv = sb[s, pl.ds(j, 16)]
                    gb[s, pl.ds(j, 16)] = sv * NR + e
                    rowid = j + lax.broadcasted_iota(jnp.int32, (16,), 0)
                    wlin[pl.ds(s * 128 + j, 16)] = plsc.load_gather(
                        crows, [rowid, e])
            pltpu.sync_copy(wlin, w_h.at[pl.ds(base, CHUNKC)])
            pltpu.sync_copy(gb, g_h.at[pl.ds(rowbase, SUBC)])
            return 0

        lax.fori_loop(0, NCHC, w_chunk, 0)


@functools.partial(
    pl.kernel,
    out_type=jax.ShapeDtypeStruct((2 * ACC_PAD, 32), jnp.float32),
    mesh=_mesh,
    scratch_types=[
        pltpu.VMEM_SHARED((ACC_PAD, 32), jnp.float32),
        pltpu.VMEM((SUBE, 128), jnp.int32),
        pltpu.VMEM((SUBE, 128), jnp.int32),
        pltpu.VMEM((CHUNKE,), jnp.float32),
        pltpu.VMEM((SUBE, 128, 32), jnp.float32),
        pltpu.VMEM((ACC_ST, 32), jnp.float32),
        pltpu.SemaphoreType.DMA,
    ],
    compiler_params=pltpu.CompilerParams(use_tc_tiling_on_sc=False),
)
def _sc_edge(tlo_h, thi_h, g_h, dst_h, w_h, acc_h,
             acc_sh, gb, db, wb, rows, stg, sem):
    cid = lax.axis_index("c")
    sid = lax.axis_index("s")

    def zfill(i, _):
        stg[i, pl.ds(0, 16)] = jnp.zeros((16,), jnp.float32)
        stg[i, pl.ds(16, 16)] = jnp.zeros((16,), jnp.float32)
        return 0

    lax.fori_loop(0, ACC_ST, zfill, 0)
    for t in range(ACC_SL // ACC_ST):
        pltpu.sync_copy(stg, acc_sh.at[pl.ds(sid * ACC_SL + t * ACC_ST,
                                             ACC_ST)])
    plsc.subcore_barrier()

    def edge_chunk(ch, _):
        rowbase = sid * (TE // 128) + ch * SUBE
        base = sid * TE + ch * CHUNKE
        pltpu.sync_copy(g_h.at[pl.ds(rowbase, SUBE)], gb)
        pltpu.sync_copy(dst_h.at[pl.ds(rowbase, SUBE)], db)
        pltpu.sync_copy(w_h.at[pl.ds(base, CHUNKE)], wb)

        @pl.when(cid == 0)
        def _():
            descs = [pltpu.async_copy(tlo_h.at[gb.at[s]], rows.at[s], sem)
                     for s in range(SUBE)]
            for dd in descs:
                dd.wait()

        @pl.when(cid == 1)
        def _():
            descs = [pltpu.async_copy(thi_h.at[gb.at[s]], rows.at[s], sem)
                     for s in range(SUBE)]
            for dd in descs:
                dd.wait()

        def scale(i, _):
            wv = wb[pl.ds(i * 16, 16)]
            for k in range(16):
                e = i * 16 + k
                s = e // 128
                j = e - s * 128
                w = wv[k]
                rows[s, j, pl.ds(0, 16)] = rows[s, j, pl.ds(0, 16)] * w
                rows[s, j, pl.ds(16, 16)] = rows[s, j, pl.ds(16, 16)] * w
            return 0

        lax.fori_loop(0, CHUNKE // 16, scale, 0)
        for s in range(SUBE):
            pltpu.sync_copy(rows.at[s], acc_sh.at[db.at[s]], add=True)
        return 0

    lax.fori_loop(0, NCHE, edge_chunk, 0)
    plsc.subcore_barrier()

    for t in range(ACC_SL // ACC_ST):
        pltpu.sync_copy(acc_sh.at[pl.ds(sid * ACC_SL + t * ACC_ST, ACC_ST)],
                        stg)
        pltpu.sync_copy(stg, acc_h.at[pl.ds(cid * ACC_PAD + sid * ACC_SL
                                            + t * ACC_ST, ACC_ST)])


def _tc_tables(ecat, b_lo, b_hi, b_u):
    def body(e_ref, lo_ref, hi_ref, u_ref, tlo_ref, thi_ref, tu_ref):
        e = e_ref[...]
        tlo_ref[...] = jnp.dot(e, lo_ref[...], preferred_element_type=jnp.float32)
        thi_ref[...] = jnp.dot(e, hi_ref[...], preferred_element_type=jnp.float32)
        tu_ref[...] = jnp.dot(e, u_ref[...], preferred_element_type=jnp.float32)

    return pl.pallas_call(
        body,
        out_shape=[
            jax.ShapeDtypeStruct((160, 96), jnp.float32),
            jax.ShapeDtypeStruct((160, 96), jnp.float32),
            jax.ShapeDtypeStruct((160, 64), jnp.float32),
        ],
    )(ecat, b_lo, b_hi, b_u)


def _tc_layer1(x, t_lo, t_hi, t_u, b1):
    def body(x_ref, tlo_ref, thi_ref, tu_ref, b_ref, olo_ref, ohi_ref, ou_ref):
        xb = x_ref[...]
        cols = lax.broadcasted_iota(jnp.int32, (NB, 160), 1)
        oh = ((cols == xb[:, 0:1]) | (cols == xb[:, 1:2] + 16)
              | (cols == xb[:, 2:3] + 32)).astype(jnp.float32)
        olo_ref[...] = jnp.dot(oh, tlo_ref[...], preferred_element_type=jnp.float32)
        ohi_ref[...] = jnp.dot(oh, thi_ref[...], preferred_element_type=jnp.float32)
        ou_ref[...] = jnp.dot(oh, tu_ref[...],
                              preferred_element_type=jnp.float32) + b_ref[...]

    return pl.pallas_call(
        body,
        grid=(NBLK,),
        in_specs=[
            pl.BlockSpec((NB, 3), lambda i: (i, 0)),
            pl.BlockSpec((160, 96), lambda i: (0, 0)),
            pl.BlockSpec((160, 96), lambda i: (0, 0)),
            pl.BlockSpec((160, 64), lambda i: (0, 0)),
            pl.BlockSpec((1, 64), lambda i: (0, 0)),
        ],
        out_specs=[
            pl.BlockSpec((NB, 96), lambda i: (i, 0)),
            pl.BlockSpec((NB, 96), lambda i: (i, 0)),
            pl.BlockSpec((NB, 64), lambda i: (i, 0)),
        ],
        out_shape=[
            jax.ShapeDtypeStruct((N, 96), jnp.float32),
            jax.ShapeDtypeStruct((N, 96), jnp.float32),
            jax.ShapeDtypeStruct((N, 64), jnp.float32),
        ],
    )(x, t_lo, t_hi, t_u, b1)


def _tc_layer2(u1, acc1, b_lo, b_hi, b_u, b2):
    def body(u_ref, a_ref, lo_ref, hi_ref, bu_ref, b2_ref,
             olo_ref, ohi_ref, ou_ref):
        h = jnp.maximum(u_ref[...] + a_ref[...], 0.0)
        olo_ref[...] = jnp.dot(h, lo_ref[...], preferred_element_type=jnp.float32)
        ohi_ref[...] = jnp.dot(h, hi_ref[...], preferred_element_type=jnp.float32)
        ou_ref[...] = jnp.dot(h, bu_ref[...],
                              preferred_element_type=jnp.float32) + b2_ref[...]

    return pl.pallas_call(
        body,
        grid=(NBLK,),
        in_specs=[
            pl.BlockSpec((NB, 64), lambda i: (i, 0)),
            pl.BlockSpec((NB, 64), lambda i: (i, 0)),
            pl.BlockSpec((64, 96), lambda i: (0, 0)),
            pl.BlockSpec((64, 96), lambda i: (0, 0)),
            pl.BlockSpec((64, 64), lambda i: (0, 0)),
            pl.BlockSpec((1, 64), lambda i: (0, 0)),
        ],
        out_specs=[
            pl.BlockSpec((NB, 96), lambda i: (i, 0)),
            pl.BlockSpec((NB, 96), lambda i: (i, 0)),
            pl.BlockSpec((NB, 64), lambda i: (i, 0)),
        ],
        out_shape=[
            jax.ShapeDtypeStruct((N, 96), jnp.float32),
            jax.ShapeDtypeStruct((N, 96), jnp.float32),
            jax.ShapeDtypeStruct((N, 64), jnp.float32),
        ],
    )(u1, acc1, b_lo, b_hi, b_u, b2)


def _tc_pool(u2, acc2, batch3):
    def body(u_ref, a_ref, b_ref, pool_ref, cnt_ref):
        @pl.when(pl.program_id(0) == 0)
        def _():
            pool_ref[...] = jnp.zeros_like(pool_ref)
            cnt_ref[...] = jnp.zeros_like(cnt_ref)

        h = jnp.maximum(u_ref[...] + a_ref[...], 0.0)
        b = b_ref[0]
        oht = (lax.broadcasted_iota(jnp.int32, (NG, NB), 0)
               == b).astype(jnp.float32)
        pool_ref[...] += jnp.dot(oht, h, preferred_element_type=jnp.float32)
        cnt_ref[...] += jnp.dot(oht, jnp.ones((NB, 8), jnp.float32),
                                preferred_element_type=jnp.float32)

    return pl.pallas_call(
        body,
        grid=(NBLK,),
        in_specs=[
            pl.BlockSpec((NB, 64), lambda i: (i, 0)),
            pl.BlockSpec((NB, 64), lambda i: (i, 0)),
            pl.BlockSpec((1, 1, NB), lambda i: (i, 0, 0)),
        ],
        out_specs=[
            pl.BlockSpec((NG, 64), lambda i: (0, 0)),
            pl.BlockSpec((NG, 8), lambda i: (0, 0)),
        ],
        out_shape=[
            jax.ShapeDtypeStruct((NG, 64), jnp.float32),
            jax.ShapeDtypeStruct((NG, 8), jnp.float32),
        ],
        compiler_params=pltpu.CompilerParams(
            dimension_semantics=("arbitrary",)),
    )(u2, acc2, batch3)


def _tc_head(pool, cnt, lin_W, lin_b):
    def body(p_ref, c_ref, w_ref, b_ref, o_ref):
        inv = 1.0 / jnp.maximum(c_ref[...][:, 0:1], 1.0)
        hg = p_ref[...] * inv
        o_ref[...] = jnp.dot(hg, w_ref[...],
                             preferred_element_type=jnp.float32) + b_ref[...]

    return pl.pallas_call(
        body,
        out_shape=jax.ShapeDtypeStruct((NG, 10), jnp.float32),
    )(pool, cnt, lin_W, lin_b)


def kernel(x, edge_index, edge_type, batch, shape_emb, color_emb, pos_emb,
           W1, root1, b1, W2, root2, b2, lin_W, lin_b):
    f32 = jnp.float32
    src = edge_index[0]
    dst = edge_index[1]

    pad = EP - E
    dst_p = jnp.concatenate([dst, jnp.full((pad,), N, jnp.int32)])
    et_p = jnp.concatenate([edge_type, jnp.zeros((pad,), jnp.int32)])
    src_p = jnp.concatenate([src, jnp.zeros((pad,), jnp.int32)])

    ecat = jnp.zeros((160, 80), f32)
    ecat = ecat.at[0:16, 0:32].set(shape_emb)
    ecat = ecat.at[16:32, 32:64].set(color_emb)
    ecat = ecat.at[32:160, 64:80].set(pos_emb)
    b1_lo = jnp.concatenate([W1[r][:, :32] for r in range(NR)], axis=1)
    b1_hi = jnp.concatenate([W1[r][:, 32:] for r in range(NR)], axis=1)
    t_lo, t_hi, t_u = _tc_tables(ecat, b1_lo, b1_hi, root1)

    dst2 = dst_p.reshape(EP // 128, 128)
    w_p, g_p = _sc_count(dst2, et_p.reshape(EP // 128, 128),
                         src_p.reshape(EP // 128, 128))

    t1lo, t1hi, u1 = _tc_layer1(x, t_lo, t_hi, t_u, b1.reshape(1, HID))
    acc1 = _sc_edge(t1lo.reshape(NR * N, 32), t1hi.reshape(NR * N, 32),
                    g_p, dst2, w_p)
    acc1c = jnp.concatenate([acc1[:N], acc1[ACC_PAD:ACC_PAD + N]], axis=1)

    b2_lo = jnp.concatenate([W2[r][:, :32] for r in range(NR)], axis=1)
    b2_hi = jnp.concatenate([W2[r][:, 32:] for r in range(NR)], axis=1)
    t2lo, t2hi, u2 = _tc_layer2(u1, acc1c, b2_lo, b2_hi, root2,
                                b2.reshape(1, HID))
    acc2 = _sc_edge(t2lo.reshape(NR * N, 32), t2hi.reshape(NR * N, 32),
                    g_p, dst2, w_p)
    acc2c = jnp.concatenate([acc2[:N], acc2[ACC_PAD:ACC_PAD + N]], axis=1)

    pool, cnt = _tc_pool(u2, acc2c, batch.reshape(NBLK, 1, NB))
    return _tc_head(pool, cnt, lin_W, lin_b.reshape(1, 10))


# speedup vs baseline: 56.2467x; 1.7295x over previous
import functools

import jax
import jax.numpy as jnp
from jax import lax
from jax.experimental import pallas as pl
from jax.experimental.pallas import tpu as pltpu
from jax.experimental.pallas import tpu_sc as plsc

N = 50000
E = 800000
NR = 3
NG = 128
HID = 64
NHALF = N // 2
CNT_ROWS = NR * NHALF
CNT_PAD = 75008
CNT_SL = CNT_PAD // 16
ACC_PAD = 50048
ACC_SL = ACC_PAD // 16
ACC_ST = 184
TE = 51200
EP = 16 * TE
SUBC = 8
CHUNKC = SUBC * 128
NCHC = TE // CHUNKC
DUMMYC = 50040
CST = 136
SUBE = 4
CHUNKE = SUBE * 128
NCHE = TE // CHUNKE
NB = 2000
NBLK = N // NB

_mesh = plsc.VectorSubcoreMesh(core_axis_name="c", subcore_axis_name="s")


@functools.partial(
    pl.kernel,
    out_type=[
        jax.ShapeDtypeStruct((EP + 128,), jnp.float32),
        jax.ShapeDtypeStruct((EP // 128, 128), jnp.int32),
    ],
    mesh=_mesh,
    scratch_types=[
        pltpu.VMEM_SHARED((ACC_PAD, 32), jnp.float32),
        pltpu.VMEM((SUBC, 128), jnp.int32),
        pltpu.VMEM((SUBC, 128), jnp.int32),
        pltpu.VMEM((SUBC, 128), jnp.int32),
        pltpu.VMEM((SUBC, 128), jnp.int32),
        pltpu.VMEM((128, 32), jnp.float32),
        pltpu.VMEM((128, 32), jnp.float32),
        pltpu.VMEM((CHUNKC,), jnp.float32),
        pltpu.VMEM((CST, 32), jnp.float32),
        pltpu.VMEM_SHARED((8, 32), jnp.float32),
    ],
    compiler_params=pltpu.CompilerParams(use_tc_tiling_on_sc=False,
                                         needs_layout_passes=False),
)
def _sc_count(dst_h, et_h, src_h, w_h, g_h,
              cnt_sh, db, eb, sb, gb, vbuf, crows, wlin, stg, oh_sh):
    cid = lax.axis_index("c")
    sid = lax.axis_index("s")

    def zfill(i, _):
        z = jnp.zeros((16,), jnp.float32)
        stg[i, pl.ds(0, 16)] = z
        stg[i, pl.ds(16, 16)] = z
        return 0

    lax.fori_loop(0, CST, zfill, 0)
    for t in range(ACC_SL // CST):
        pltpu.sync_copy(stg, cnt_sh.at[pl.ds(sid * ACC_SL + t * CST, CST)])

    iot = lax.broadcasted_iota(jnp.int32, (16,), 0)

    @pl.when(sid == 0)
    def _():
        stg[0, pl.ds(0, 16)] = jnp.where(iot == 0, 1.0, 0.0)
        stg[1, pl.ds(0, 16)] = jnp.where(iot == 1, 1.0, 0.0)
        stg[2, pl.ds(0, 16)] = jnp.where(iot == 2, 1.0, 0.0)
        pltpu.sync_copy(stg.at[pl.ds(0, 8)], oh_sh)

    plsc.subcore_barrier()

    def count_chunk(ch, _):
        rowbase = sid * (TE // 128) + ch * SUBC
        pltpu.sync_copy(dst_h.at[pl.ds(rowbase, SUBC)], db)
        pltpu.sync_copy(et_h.at[pl.ds(rowbase, SUBC)], eb)
        for s in range(SUBC):
            pltpu.sync_copy(oh_sh.at[eb.at[s]], vbuf)
            pltpu.sync_copy(vbuf, cnt_sh.at[db.at[s]], add=True)
        return 0

    lax.fori_loop(0, NCHC, count_chunk, 0)
    plsc.subcore_barrier()

    def inv_blk(t, _):
        nb = sid * ACC_SL + t * CST
        pltpu.sync_copy(cnt_sh.at[pl.ds(nb, CST)], stg)

        def inv_row(u, _):
            v = stg[u, pl.ds(0, 16)]
            stg[u, pl.ds(0, 16)] = 1.0 / jnp.maximum(v, 1.0)
            return 0

        lax.fori_loop(0, CST, inv_row, 0)
        pltpu.sync_copy(stg, cnt_sh.at[pl.ds(nb, CST)])
        return 0

    lax.fori_loop(0, ACC_SL // CST, inv_blk, 0)
    plsc.subcore_barrier()

    def w_chunk(ch, _):
        rowbase = sid * (TE // 128) + ch * SUBC
        base = sid * TE + ch * CHUNKC
        pltpu.sync_copy(dst_h.at[pl.ds(rowbase, SUBC)], db)
        pltpu.sync_copy(et_h.at[pl.ds(rowbase, SUBC)], eb)
        pltpu.sync_copy(src_h.at[pl.ds(rowbase, SUBC)], sb)
        for s in range(SUBC):
            pltpu.sync_copy(cnt_sh.at[db.at[s]], crows)
            for i in range(8):
                j = i * 16
                e = eb[s, pl.ds(j, 16)]
                sv = sb[s, pl.ds(j, 16)]
                gb[s, pl.ds(j, 16)] = sv * NR + e
                rowid = j + lax.broadcasted_iota(jnp.int32, (16,), 0)
                wlin[pl.ds(s * 128 + j, 16)] = plsc.load_gather(
                    crows, [rowid, e])
        pltpu.sync_copy(wlin, w_h.at[pl.ds(base, CHUNKC)])
        pltpu.sync_copy(gb, g_h.at[pl.ds(rowbase, SUBC)])
        return 0

    lax.fori_loop(cid * (NCHC // 2), (cid + 1) * (NCHC // 2), w_chunk, 0)


@functools.partial(
    pl.kernel,
    out_type=jax.ShapeDtypeStruct((2 * ACC_PAD, 32), jnp.float32),
    mesh=_mesh,
    scratch_types=[
        pltpu.VMEM_SHARED((ACC_PAD, 32), jnp.float32),
        pltpu.VMEM((SUBE, 128), jnp.int32),
        pltpu.VMEM((SUBE, 128), jnp.int32),
        pltpu.VMEM((CHUNKE,), jnp.float32),
        pltpu.VMEM((SUBE, 128, 32), jnp.float32),
        pltpu.VMEM((ACC_ST, 32), jnp.float32),
        pltpu.SemaphoreType.DMA,
    ],
    compiler_params=pltpu.CompilerParams(use_tc_tiling_on_sc=False),
)
def _sc_edge(tlo_h, thi_h, g_h, dst_h, w_h, acc_h,
             acc_sh, gb, db, wb, rows, stg, sem):
    cid = lax.axis_index("c")
    sid = lax.axis_index("s")

    def zfill(i, _):
        stg[i, pl.ds(0, 16)] = jnp.zeros((16,), jnp.float32)
        stg[i, pl.ds(16, 16)] = jnp.zeros((16,), jnp.float32)
        return 0

    lax.fori_loop(0, ACC_ST, zfill, 0)
    for t in range(ACC_SL // ACC_ST):
        pltpu.sync_copy(stg, acc_sh.at[pl.ds(sid * ACC_SL + t * ACC_ST,
                                             ACC_ST)])
    plsc.subcore_barrier()

    def edge_chunk(ch, _):
        rowbase = sid * (TE // 128) + ch * SUBE
        base = sid * TE + ch * CHUNKE
        pltpu.sync_copy(g_h.at[pl.ds(rowbase, SUBE)], gb)
        pltpu.sync_copy(dst_h.at[pl.ds(rowbase, SUBE)], db)
        pltpu.sync_copy(w_h.at[pl.ds(base, CHUNKE)], wb)

        @pl.when(cid == 0)
        def _():
            descs = [pltpu.async_copy(tlo_h.at[gb.at[s]], rows.at[s], sem)
                     for s in range(SUBE)]
            for dd in descs:
                dd.wait()

        @pl.when(cid == 1)
        def _():
            descs = [pltpu.async_copy(thi_h.at[gb.at[s]], rows.at[s], sem)
                     for s in range(SUBE)]
            for dd in descs:
                dd.wait()

        def scale(i, _):
            wv = wb[pl.ds(i * 16, 16)]
            for k in range(16):
                e = i * 16 + k
                s = e // 128
                j = e - s * 128
                w = wv[k]
                rows[s, j, pl.ds(0, 16)] = rows[s, j, pl.ds(0, 16)] * w
                rows[s, j, pl.ds(16, 16)] = rows[s, j, pl.ds(16, 16)] * w
            return 0

        lax.fori_loop(0, CHUNKE // 16, scale, 0)
        for s in range(SUBE):
            pltpu.sync_copy(rows.at[s], acc_sh.at[db.at[s]], add=True)
        return 0

    lax.fori_loop(0, NCHE, edge_chunk, 0)
    plsc.subcore_barrier()

    for t in range(ACC_SL // ACC_ST):
        pltpu.sync_copy(acc_sh.at[pl.ds(sid * ACC_SL + t * ACC_ST, ACC_ST)],
                        stg)
        pltpu.sync_copy(stg, acc_h.at[pl.ds(cid * ACC_PAD + sid * ACC_SL
                                            + t * ACC_ST, ACC_ST)])


def _tc_tables(ecat, b_lo, b_hi, b_u):
    def body(e_ref, lo_ref, hi_ref, u_ref, tlo_ref, thi_ref, tu_ref):
        e = e_ref[...]
        tlo_ref[...] = jnp.dot(e, lo_ref[...], preferred_element_type=jnp.float32)
        thi_ref[...] = jnp.dot(e, hi_ref[...], preferred_element_type=jnp.float32)
        tu_ref[...] = jnp.dot(e, u_ref[...], preferred_element_type=jnp.float32)

    return pl.pallas_call(
        body,
        out_shape=[
            jax.ShapeDtypeStruct((160, 96), jnp.float32),
            jax.ShapeDtypeStruct((160, 96), jnp.float32),
            jax.ShapeDtypeStruct((160, 64), jnp.float32),
        ],
    )(ecat, b_lo, b_hi, b_u)


def _tc_layer1(x, t_lo, t_hi, t_u, b1):
    def body(x_ref, tlo_ref, thi_ref, tu_ref, b_ref, olo_ref, ohi_ref, ou_ref):
        xb = x_ref[...]
        cols = lax.broadcasted_iota(jnp.int32, (NB, 160), 1)
        oh = ((cols == xb[:, 0:1]) | (cols == xb[:, 1:2] + 16)
              | (cols == xb[:, 2:3] + 32)).astype(jnp.float32)
        olo_ref[...] = jnp.dot(oh, tlo_ref[...], preferred_element_type=jnp.float32)
        ohi_ref[...] = jnp.dot(oh, thi_ref[...], preferred_element_type=jnp.float32)
        ou_ref[...] = jnp.dot(oh, tu_ref[...],
                              preferred_element_type=jnp.float32) + b_ref[...]

    return pl.pallas_call(
        body,
        grid=(NBLK,),
        in_specs=[
            pl.BlockSpec((NB, 3), lambda i: (i, 0)),
            pl.BlockSpec((160, 96), lambda i: (0, 0)),
            pl.BlockSpec((160, 96), lambda i: (0, 0)),
            pl.BlockSpec((160, 64), lambda i: (0, 0)),
            pl.BlockSpec((1, 64), lambda i: (0, 0)),
        ],
        out_specs=[
            pl.BlockSpec((NB, 96), lambda i: (i, 0)),
            pl.BlockSpec((NB, 96), lambda i: (i, 0)),
            pl.BlockSpec((NB, 64), lambda i: (i, 0)),
        ],
        out_shape=[
            jax.ShapeDtypeStruct((N, 96), jnp.float32),
            jax.ShapeDtypeStruct((N, 96), jnp.float32),
            jax.ShapeDtypeStruct((N, 64), jnp.float32),
        ],
    )(x, t_lo, t_hi, t_u, b1)


def _tc_layer2(u1, acc1, b_lo, b_hi, b_u, b2):
    def body(u_ref, a_ref, lo_ref, hi_ref, bu_ref, b2_ref,
             olo_ref, ohi_ref, ou_ref):
        h = jnp.maximum(u_ref[...] + a_ref[...], 0.0)
        olo_ref[...] = jnp.dot(h, lo_ref[...], preferred_element_type=jnp.float32)
        ohi_ref[...] = jnp.dot(h, hi_ref[...], preferred_element_type=jnp.float32)
        ou_ref[...] = jnp.dot(h, bu_ref[...],
                              preferred_element_type=jnp.float32) + b2_ref[...]

    return pl.pallas_call(
        body,
        grid=(NBLK,),
        in_specs=[
            pl.BlockSpec((NB, 64), lambda i: (i, 0)),
            pl.BlockSpec((NB, 64), lambda i: (i, 0)),
            pl.BlockSpec((64, 96), lambda i: (0, 0)),
            pl.BlockSpec((64, 96), lambda i: (0, 0)),
            pl.BlockSpec((64, 64), lambda i: (0, 0)),
            pl.BlockSpec((1, 64), lambda i: (0, 0)),
        ],
        out_specs=[
            pl.BlockSpec((NB, 96), lambda i: (i, 0)),
            pl.BlockSpec((NB, 96), lambda i: (i, 0)),
            pl.BlockSpec((NB, 64), lambda i: (i, 0)),
        ],
        out_shape=[
            jax.ShapeDtypeStruct((N, 96), jnp.float32),
            jax.ShapeDtypeStruct((N, 96), jnp.float32),
            jax.ShapeDtypeStruct((N, 64), jnp.float32),
        ],
    )(u1, acc1, b_lo, b_hi, b_u, b2)


def _tc_pool(u2, acc2, batch3):
    def body(u_ref, a_ref, b_ref, pool_ref, cnt_ref):
        @pl.when(pl.program_id(0) == 0)
        def _():
            pool_ref[...] = jnp.zeros_like(pool_ref)
            cnt_ref[...] = jnp.zeros_like(cnt_ref)

        h = jnp.maximum(u_ref[...] + a_ref[...], 0.0)
        b = b_ref[0]
        oht = (lax.broadcasted_iota(jnp.int32, (NG, NB), 0)
               == b).astype(jnp.float32)
        pool_ref[...] += jnp.dot(oht, h, preferred_element_type=jnp.float32)
        cnt_ref[...] += jnp.dot(oht, jnp.ones((NB, 8), jnp.float32),
                                preferred_element_type=jnp.float32)

    return pl.pallas_call(
        body,
        grid=(NBLK,),
        in_specs=[
            pl.BlockSpec((NB, 64), lambda i: (i, 0)),
            pl.BlockSpec((NB, 64), lambda i: (i, 0)),
            pl.BlockSpec((1, 1, NB), lambda i: (i, 0, 0)),
        ],
        out_specs=[
            pl.BlockSpec((NG, 64), lambda i: (0, 0)),
            pl.BlockSpec((NG, 8), lambda i: (0, 0)),
        ],
        out_shape=[
            jax.ShapeDtypeStruct((NG, 64), jnp.float32),
            jax.ShapeDtypeStruct((NG, 8), jnp.float32),
        ],
        compiler_params=pltpu.CompilerParams(
            dimension_semantics=("arbitrary",)),
    )(u2, acc2, batch3)


def _tc_head(pool, cnt, lin_W, lin_b):
    def body(p_ref, c_ref, w_ref, b_ref, o_ref):
        inv = 1.0 / jnp.maximum(c_ref[...][:, 0:1], 1.0)
        hg = p_ref[...] * inv
        o_ref[...] = jnp.dot(hg, w_ref[...],
                             preferred_element_type=jnp.float32) + b_ref[...]

    return pl.pallas_call(
        body,
        out_shape=jax.ShapeDtypeStruct((NG, 10), jnp.float32),
    )(pool, cnt, lin_W, lin_b)


def kernel(x, edge_index, edge_type, batch, shape_emb, color_emb, pos_emb,
           W1, root1, b1, W2, root2, b2, lin_W, lin_b):
    f32 = jnp.float32
    src = edge_index[0]
    dst = edge_index[1]

    pad = EP - E
    dst_p = jnp.concatenate([dst, jnp.full((pad,), N, jnp.int32)])
    et_p = jnp.concatenate([edge_type, jnp.zeros((pad,), jnp.int32)])
    src_p = jnp.concatenate([src, jnp.zeros((pad,), jnp.int32)])

    ecat = jnp.zeros((160, 80), f32)
    ecat = ecat.at[0:16, 0:32].set(shape_emb)
    ecat = ecat.at[16:32, 32:64].set(color_emb)
    ecat = ecat.at[32:160, 64:80].set(pos_emb)
    b1_lo = jnp.concatenate([W1[r][:, :32] for r in range(NR)], axis=1)
    b1_hi = jnp.concatenate([W1[r][:, 32:] for r in range(NR)], axis=1)
    t_lo, t_hi, t_u = _tc_tables(ecat, b1_lo, b1_hi, root1)

    dst2 = dst_p.reshape(EP // 128, 128)
    w_p, g_p = _sc_count(dst2, et_p.reshape(EP // 128, 128),
                         src_p.reshape(EP // 128, 128))

    t1lo, t1hi, u1 = _tc_layer1(x, t_lo, t_hi, t_u, b1.reshape(1, HID))
    acc1 = _sc_edge(t1lo.reshape(NR * N, 32), t1hi.reshape(NR * N, 32),
                    g_p, dst2, w_p)
    acc1c = jnp.concatenate([acc1[:N], acc1[ACC_PAD:ACC_PAD + N]], axis=1)

    b2_lo = jnp.concatenate([W2[r][:, :32] for r in range(NR)], axis=1)
    b2_hi = jnp.concatenate([W2[r][:, 32:] for r in range(NR)], axis=1)
    t2lo, t2hi, u2 = _tc_layer2(u1, acc1c, b2_lo, b2_hi, root2,
                                b2.reshape(1, HID))
    acc2 = _sc_edge(t2lo.reshape(NR * N, 32), t2hi.reshape(NR * N, 32),
                    g_p, dst2, w_p)
    acc2c = jnp.concatenate([acc2[:N], acc2[ACC_PAD:ACC_PAD + N]], axis=1)

    pool, cnt = _tc_pool(u2, acc2c, batch.reshape(NBLK, 1, NB))
    return _tc_head(pool, cnt, lin_W, lin_b.reshape(1, 10))


# speedup vs baseline: 59.8355x; 1.0638x over previous
import functools

import jax
import jax.numpy as jnp
from jax import lax
from jax.experimental import pallas as pl
from jax.experimental.pallas import tpu as pltpu
from jax.experimental.pallas import tpu_sc as plsc

N = 50000
E = 800000
NR = 3
NG = 128
HID = 64
NHALF = N // 2
CNT_ROWS = NR * NHALF
CNT_PAD = 75008
CNT_SL = CNT_PAD // 16
ACC_PAD = 50048
ACC_SL = ACC_PAD // 16
ACC_ST = 184
TE = 51200
EP = 16 * TE
SUBC = 8
CHUNKC = SUBC * 128
NCHC = TE // CHUNKC
DUMMYC = 50040
CST = 136
SUBE = 4
CHUNKE = SUBE * 128
NCHE = TE // CHUNKE
NB = 2000
NBLK = N // NB

_mesh = plsc.VectorSubcoreMesh(core_axis_name="c", subcore_axis_name="s")


@functools.partial(
    pl.kernel,
    out_type=[
        jax.ShapeDtypeStruct((EP + 128,), jnp.float32),
        jax.ShapeDtypeStruct((EP // 128, 128), jnp.int32),
    ],
    mesh=_mesh,
    scratch_types=[
        pltpu.VMEM_SHARED((ACC_PAD, 32), jnp.float32),
        pltpu.VMEM((SUBC, 128), jnp.int32),
        pltpu.VMEM((SUBC, 128), jnp.int32),
        pltpu.VMEM((SUBC, 128), jnp.int32),
        pltpu.VMEM((SUBC, 128), jnp.int32),
        pltpu.VMEM((128, 32), jnp.float32),
        pltpu.VMEM((128, 32), jnp.float32),
        pltpu.VMEM((CHUNKC,), jnp.float32),
        pltpu.VMEM((CST, 32), jnp.float32),
        pltpu.VMEM_SHARED((8, 32), jnp.float32),
    ],
    compiler_params=pltpu.CompilerParams(use_tc_tiling_on_sc=False,
                                         needs_layout_passes=False),
)
def _sc_count(dst_h, et_h, src_h, w_h, g_h,
              cnt_sh, db, eb, sb, gb, vbuf, crows, wlin, stg, oh_sh):
    cid = lax.axis_index("c")
    sid = lax.axis_index("s")

    def zfill(i, _):
        z = jnp.zeros((16,), jnp.float32)
        stg[i, pl.ds(0, 16)] = z
        stg[i, pl.ds(16, 16)] = z
        return 0

    lax.fori_loop(0, CST, zfill, 0)
    for t in range(ACC_SL // CST):
        pltpu.sync_copy(stg, cnt_sh.at[pl.ds(sid * ACC_SL + t * CST, CST)])

    iot = lax.broadcasted_iota(jnp.int32, (16,), 0)

    @pl.when(sid == 0)
    def _():
        stg[0, pl.ds(0, 16)] = jnp.where(iot == 0, 1.0, 0.0)
        stg[1, pl.ds(0, 16)] = jnp.where(iot == 1, 1.0, 0.0)
        stg[2, pl.ds(0, 16)] = jnp.where(iot == 2, 1.0, 0.0)
        pltpu.sync_copy(stg.at[pl.ds(0, 8)], oh_sh)

    plsc.subcore_barrier()

    def count_chunk(ch, _):
        rowbase = sid * (TE // 128) + ch * SUBC
        pltpu.sync_copy(dst_h.at[pl.ds(rowbase, SUBC)], db)
        pltpu.sync_copy(et_h.at[pl.ds(rowbase, SUBC)], eb)
        for s in range(SUBC):
            pltpu.sync_copy(oh_sh.at[eb.at[s]], vbuf)
            pltpu.sync_copy(vbuf, cnt_sh.at[db.at[s]], add=True)
        return 0

    lax.fori_loop(0, NCHC, count_chunk, 0)
    plsc.subcore_barrier()

    def inv_blk(t, _):
        nb = sid * ACC_SL + t * CST
        pltpu.sync_copy(cnt_sh.at[pl.ds(nb, CST)], stg)

        def inv_row(u, _):
            v = stg[u, pl.ds(0, 16)]
            stg[u, pl.ds(0, 16)] = 1.0 / jnp.maximum(v, 1.0)
            return 0

        lax.fori_loop(0, CST, inv_row, 0)
        pltpu.sync_copy(stg, cnt_sh.at[pl.ds(nb, CST)])
        return 0

    lax.fori_loop(0, ACC_SL // CST, inv_blk, 0)
    plsc.subcore_barrier()

    def w_chunk(ch, _):
        rowbase = sid * (TE // 128) + ch * SUBC
        base = sid * TE + ch * CHUNKC
        pltpu.sync_copy(dst_h.at[pl.ds(rowbase, SUBC)], db)
        pltpu.sync_copy(et_h.at[pl.ds(rowbase, SUBC)], eb)
        pltpu.sync_copy(src_h.at[pl.ds(rowbase, SUBC)], sb)
        for s in range(SUBC):
            pltpu.sync_copy(cnt_sh.at[db.at[s]], crows)
            for i in range(8):
                j = i * 16
                e = eb[s, pl.ds(j, 16)]
                sv = sb[s, pl.ds(j, 16)]
                gb[s, pl.ds(j, 16)] = sv * NR + e
                rowid = j + lax.broadcasted_iota(jnp.int32, (16,), 0)
                wlin[pl.ds(s * 128 + j, 16)] = plsc.load_gather(
                    crows, [rowid, e])
        pltpu.sync_copy(wlin, w_h.at[pl.ds(base, CHUNKC)])
        pltpu.sync_copy(gb, g_h.at[pl.ds(rowbase, SUBC)])
        return 0

    lax.fori_loop(cid * (NCHC // 2), (cid + 1) * (NCHC // 2), w_chunk, 0)


@functools.partial(
    pl.kernel,
    out_type=jax.ShapeDtypeStruct((2 * ACC_PAD, 32), jnp.float32),
    mesh=_mesh,
    scratch_types=[
        pltpu.VMEM_SHARED((ACC_PAD, 32), jnp.float32),
        pltpu.VMEM((SUBE, 128), jnp.int32),
        pltpu.VMEM((SUBE, 128), jnp.int32),
        pltpu.VMEM((CHUNKE,), jnp.float32),
        pltpu.VMEM((SUBE, 128, 32), jnp.float32),
        pltpu.VMEM((ACC_ST, 32), jnp.float32),
        pltpu.SemaphoreType.DMA,
    ],
    compiler_params=pltpu.CompilerParams(use_tc_tiling_on_sc=False),
)
def _sc_edge(tlo_h, thi_h, g_h, dst_h, w_h, acc_h,
             acc_sh, gb, db, wb, rows, stg, sem):
    cid = lax.axis_index("c")
    sid = lax.axis_index("s")

    def zfill(i, _):
        stg[i, pl.ds(0, 16)] = jnp.zeros((16,), jnp.float32)
        stg[i, pl.ds(16, 16)] = jnp.zeros((16,), jnp.float32)
        return 0

    lax.fori_loop(0, ACC_ST, zfill, 0)
    for t in range(ACC_SL // ACC_ST):
        pltpu.sync_copy(stg, acc_sh.at[pl.ds(sid * ACC_SL + t * ACC_ST,
                                             ACC_ST)])
    plsc.subcore_barrier()

    def edge_chunk(ch, _):
        rowbase = sid * (TE // 128) + ch * SUBE
        base = sid * TE + ch * CHUNKE
        pltpu.sync_copy(g_h.at[pl.ds(rowbase, SUBE)], gb)
        pltpu.sync_copy(dst_h.at[pl.ds(rowbase, SUBE)], db)
        pltpu.sync_copy(w_h.at[pl.ds(base, CHUNKE)], wb)

        def run(t_h):
            descs = [pltpu.async_copy(t_h.at[gb.at[s]], rows.at[s], sem)
                     for s in range(SUBE)]
            for s in range(SUBE):
                descs[s].wait()

                def scale(i, _):
                    wv = wb[pl.ds(s * 128 + i * 16, 16)]
                    for k in range(16):
                        j = i * 16 + k
                        w = wv[k]
                        rows[s, j, pl.ds(0, 16)] = rows[s, j, pl.ds(0, 16)] * w
                        rows[s, j, pl.ds(16, 16)] = (rows[s, j, pl.ds(16, 16)]
                                                     * w)
                    return 0

                lax.fori_loop(0, 8, scale, 0)
                pltpu.sync_copy(rows.at[s], acc_sh.at[db.at[s]], add=True)

        @pl.when(cid == 0)
        def _():
            run(tlo_h)

        @pl.when(cid == 1)
        def _():
            run(thi_h)

        return 0

    lax.fori_loop(0, NCHE, edge_chunk, 0)
    plsc.subcore_barrier()

    for t in range(ACC_SL // ACC_ST):
        pltpu.sync_copy(acc_sh.at[pl.ds(sid * ACC_SL + t * ACC_ST, ACC_ST)],
                        stg)
        pltpu.sync_copy(stg, acc_h.at[pl.ds(cid * ACC_PAD + sid * ACC_SL
                                            + t * ACC_ST, ACC_ST)])


def _tc_tables(ecat, b_lo, b_hi, b_u):
    def body(e_ref, lo_ref, hi_ref, u_ref, tlo_ref, thi_ref, tu_ref):
        e = e_ref[...]
        tlo_ref[...] = jnp.dot(e, lo_ref[...], preferred_element_type=jnp.float32)
        thi_ref[...] = jnp.dot(e, hi_ref[...], preferred_element_type=jnp.float32)
        tu_ref[...] = jnp.dot(e, u_ref[...], preferred_element_type=jnp.float32)

    return pl.pallas_call(
        body,
        out_shape=[
            jax.ShapeDtypeStruct((160, 96), jnp.float32),
            jax.ShapeDtypeStruct((160, 96), jnp.float32),
            jax.ShapeDtypeStruct((160, 64), jnp.float32),
        ],
    )(ecat, b_lo, b_hi, b_u)


def _tc_layer1(x, t_lo, t_hi, t_u, b1):
    def body(x_ref, tlo_ref, thi_ref, tu_ref, b_ref, olo_ref, ohi_ref, ou_ref):
        xb = x_ref[...]
        cols = lax.broadcasted_iota(jnp.int32, (NB, 160), 1)
        oh = ((cols == xb[:, 0:1]) | (cols == xb[:, 1:2] + 16)
              | (cols == xb[:, 2:3] + 32)).astype(jnp.float32)
        olo_ref[...] = jnp.dot(oh, tlo_ref[...], preferred_element_type=jnp.float32)
        ohi_ref[...] = jnp.dot(oh, thi_ref[...], preferred_element_type=jnp.float32)
        ou_ref[...] = jnp.dot(oh, tu_ref[...],
                              preferred_element_type=jnp.float32) + b_ref[...]

    return pl.pallas_call(
        body,
        grid=(NBLK,),
        in_specs=[
            pl.BlockSpec((NB, 3), lambda i: (i, 0)),
            pl.BlockSpec((160, 96), lambda i: (0, 0)),
            pl.BlockSpec((160, 96), lambda i: (0, 0)),
            pl.BlockSpec((160, 64), lambda i: (0, 0)),
            pl.BlockSpec((1, 64), lambda i: (0, 0)),
        ],
        out_specs=[
            pl.BlockSpec((NB, 96), lambda i: (i, 0)),
            pl.BlockSpec((NB, 96), lambda i: (i, 0)),
            pl.BlockSpec((NB, 64), lambda i: (i, 0)),
        ],
        out_shape=[
            jax.ShapeDtypeStruct((N, 96), jnp.float32),
            jax.ShapeDtypeStruct((N, 96), jnp.float32),
            jax.ShapeDtypeStruct((N, 64), jnp.float32),
        ],
    )(x, t_lo, t_hi, t_u, b1)


def _tc_layer2(u1, acc1, b_lo, b_hi, b_u, b2):
    def body(u_ref, a_ref, lo_ref, hi_ref, bu_ref, b2_ref,
             olo_ref, ohi_ref, ou_ref):
        h = jnp.maximum(u_ref[...] + a_ref[...], 0.0)
        olo_ref[...] = jnp.dot(h, lo_ref[...], preferred_element_type=jnp.float32)
        ohi_ref[...] = jnp.dot(h, hi_ref[...], preferred_element_type=jnp.float32)
        ou_ref[...] = jnp.dot(h, bu_ref[...],
                              preferred_element_type=jnp.float32) + b2_ref[...]

    return pl.pallas_call(
        body,
        grid=(NBLK,),
        in_specs=[
            pl.BlockSpec((NB, 64), lambda i: (i, 0)),
            pl.BlockSpec((NB, 64), lambda i: (i, 0)),
            pl.BlockSpec((64, 96), lambda i: (0, 0)),
            pl.BlockSpec((64, 96), lambda i: (0, 0)),
            pl.BlockSpec((64, 64), lambda i: (0, 0)),
            pl.BlockSpec((1, 64), lambda i: (0, 0)),
        ],
        out_specs=[
            pl.BlockSpec((NB, 96), lambda i: (i, 0)),
            pl.BlockSpec((NB, 96), lambda i: (i, 0)),
            pl.BlockSpec((NB, 64), lambda i: (i, 0)),
        ],
        out_shape=[
            jax.ShapeDtypeStruct((N, 96), jnp.float32),
            jax.ShapeDtypeStruct((N, 96), jnp.float32),
            jax.ShapeDtypeStruct((N, 64), jnp.float32),
        ],
    )(u1, acc1, b_lo, b_hi, b_u, b2)


def _tc_pool(u2, acc2, batch3):
    def body(u_ref, a_ref, b_ref, pool_ref, cnt_ref):
        @pl.when(pl.program_id(0) == 0)
        def _():
            pool_ref[...] = jnp.zeros_like(pool_ref)
            cnt_ref[...] = jnp.zeros_like(cnt_ref)

        h = jnp.maximum(u_ref[...] + a_ref[...], 0.0)
        b = b_ref[0]
        oht = (lax.broadcasted_iota(jnp.int32, (NG, NB), 0)
               == b).astype(jnp.float32)
        pool_ref[...] += jnp.dot(oht, h, preferred_element_type=jnp.float32)
        cnt_ref[...] += jnp.dot(oht, jnp.ones((NB, 8), jnp.float32),
                                preferred_element_type=jnp.float32)

    return pl.pallas_call(
        body,
        grid=(NBLK,),
        in_specs=[
            pl.BlockSpec((NB, 64), lambda i: (i, 0)),
            pl.BlockSpec((NB, 64), lambda i: (i, 0)),
            pl.BlockSpec((1, 1, NB), lambda i: (i, 0, 0)),
        ],
        out_specs=[
            pl.BlockSpec((NG, 64), lambda i: (0, 0)),
            pl.BlockSpec((NG, 8), lambda i: (0, 0)),
        ],
        out_shape=[
            jax.ShapeDtypeStruct((NG, 64), jnp.float32),
            jax.ShapeDtypeStruct((NG, 8), jnp.float32),
        ],
        compiler_params=pltpu.CompilerParams(
            dimension_semantics=("arbitrary",)),
    )(u2, acc2, batch3)


def _tc_head(pool, cnt, lin_W, lin_b):
    def body(p_ref, c_ref, w_ref, b_ref, o_ref):
        inv = 1.0 / jnp.maximum(c_ref[...][:, 0:1], 1.0)
        hg = p_ref[...] * inv
        o_ref[...] = jnp.dot(hg, w_ref[...],
                             preferred_element_type=jnp.float32) + b_ref[...]

    return pl.pallas_call(
        body,
        out_shape=jax.ShapeDtypeStruct((NG, 10), jnp.float32),
    )(pool, cnt, lin_W, lin_b)


def kernel(x, edge_index, edge_type, batch, shape_emb, color_emb, pos_emb,
           W1, root1, b1, W2, root2, b2, lin_W, lin_b):
    f32 = jnp.float32
    src = edge_index[0]
    dst = edge_index[1]

    pad = EP - E
    dst_p = jnp.concatenate([dst, jnp.full((pad,), N, jnp.int32)])
    et_p = jnp.concatenate([edge_type, jnp.zeros((pad,), jnp.int32)])
    src_p = jnp.concatenate([src, jnp.zeros((pad,), jnp.int32)])

    ecat = jnp.zeros((160, 80), f32)
    ecat = ecat.at[0:16, 0:32].set(shape_emb)
    ecat = ecat.at[16:32, 32:64].set(color_emb)
    ecat = ecat.at[32:160, 64:80].set(pos_emb)
    b1_lo = jnp.concatenate([W1[r][:, :32] for r in range(NR)], axis=1)
    b1_hi = jnp.concatenate([W1[r][:, 32:] for r in range(NR)], axis=1)
    t_lo, t_hi, t_u = _tc_tables(ecat, b1_lo, b1_hi, root1)

    dst2 = dst_p.reshape(EP // 128, 128)
    w_p, g_p = _sc_count(dst2, et_p.reshape(EP // 128, 128),
                         src_p.reshape(EP // 128, 128))

    t1lo, t1hi, u1 = _tc_layer1(x, t_lo, t_hi, t_u, b1.reshape(1, HID))
    acc1 = _sc_edge(t1lo.reshape(NR * N, 32), t1hi.reshape(NR * N, 32),
                    g_p, dst2, w_p)
    acc1c = jnp.concatenate([acc1[:N], acc1[ACC_PAD:ACC_PAD + N]], axis=1)

    b2_lo = jnp.concatenate([W2[r][:, :32] for r in range(NR)], axis=1)
    b2_hi = jnp.concatenate([W2[r][:, 32:] for r in range(NR)], axis=1)
    t2lo, t2hi, u2 = _tc_layer2(u1, acc1c, b2_lo, b2_hi, root2,
                                b2.reshape(1, HID))
    acc2 = _sc_edge(t2lo.reshape(NR * N, 32), t2hi.reshape(NR * N, 32),
                    g_p, dst2, w_p)
    acc2c = jnp.concatenate([acc2[:N], acc2[ACC_PAD:ACC_PAD + N]], axis=1)

    pool, cnt = _tc_pool(u2, acc2c, batch.reshape(NBLK, 1, NB))
    return _tc_head(pool, cnt, lin_W, lin_b.reshape(1, 10))
